# baseline (speedup 1.0000x reference)
import functools

import jax
import jax.numpy as jnp
from jax import lax
from jax.experimental import pallas as pl
from jax.experimental.pallas import tpu as pltpu
from jax.experimental.pallas import tpu_sc as plsc

E = 320000
A = 640000
D = 128
INT = 64


_BE = 2560
_BA = 2560


def _k1_body(m_ref, e_ref, wm_ref, b_ref, we1_ref, we2_ref, mt_ref, et_ref):
    z = jnp.dot(m_ref[...], wm_ref[...], preferred_element_type=jnp.float32)
    z = z + b_ref[...]
    mt_ref[...] = z * jax.nn.sigmoid(z)
    e1 = jnp.dot(e_ref[...], we1_ref[...], preferred_element_type=jnp.float32)
    et_ref[...] = jnp.dot(e1, we2_ref[...], preferred_element_type=jnp.float32)


def _k3_body(h_ref, a_ref, wa1_ref, wa2_ref, wd_ref, em_ref):
    s = jnp.dot(h_ref[...], wd_ref[...], preferred_element_type=jnp.float32)
    s = s * jax.nn.sigmoid(s)
    a1 = jnp.dot(a_ref[...], wa1_ref[...], preferred_element_type=jnp.float32)
    ap = jnp.dot(a1, wa2_ref[...], preferred_element_type=jnp.float32)
    em_ref[...] = s * ap


def _k5_body(g_ref, wu_ref, o_ref):
    z = jnp.dot(g_ref[...], wu_ref[...], preferred_element_type=jnp.float32)
    o_ref[...] = z * jax.nn.sigmoid(z)


def _row_blocked(nrows, block, ncols):
    return pl.BlockSpec((block, ncols), lambda i: (i, 0))


def _full(shape):
    return pl.BlockSpec(shape, lambda i: tuple(0 for _ in shape))


_NW = 32
_B2 = 256
_NBLK2 = A // _B2


def _k2_body(mt_hbm, et_hbm, kj2_hbm, ji2_hbm, h_hbm,
             kjb, jib, mrows, erows, sem):
    c = lax.axis_index("c")
    s = lax.axis_index("s")
    wid = s * 2 + c
    nfull = _NBLK2 // _NW
    rem = _NBLK2 % _NW
    trips = jnp.where(wid < rem, nfull + 1, nfull)

    @pl.loop(0, trips)
    def _block(g):
        blk = wid + g * _NW
        base = blk * _B2
        pltpu.sync_copy(kj2_hbm.at[pl.ds(blk * 2, 2)], kjb)
        pltpu.sync_copy(ji2_hbm.at[pl.ds(blk * 2, 2)], jib)
        descs = []
        for j in range(2):
            descs.append(pltpu.async_copy(
                mt_hbm.at[kjb.at[j]], mrows.at[pl.ds(j * 128, 128)], sem))
            descs.append(pltpu.async_copy(
                et_hbm.at[jib.at[j]], erows.at[pl.ds(j * 128, 128)], sem))
        for d in descs:
            d.wait()

        @pl.loop(0, _B2)
        def _row(r):
            for cc in range(D // 16):
                sl = pl.ds(cc * 16, 16)
                mrows[r, sl] = mrows[r, sl] * erows[r, sl]

        pltpu.sync_copy(mrows, h_hbm.at[pl.ds(base, _B2)])


def _k2_call(mt, et, kj2, ji2):
    mesh = plsc.VectorSubcoreMesh(core_axis_name="c", subcore_axis_name="s")
    f = pl.kernel(
        _k2_body,
        out_type=jax.ShapeDtypeStruct((A, D), jnp.float32),
        mesh=mesh,
        compiler_params=pltpu.CompilerParams(needs_layout_passes=False),
        scratch_types=[
            pltpu.VMEM((2, 128), jnp.int32),
            pltpu.VMEM((2, 128), jnp.int32),
            pltpu.VMEM((_B2, D), jnp.float32),
            pltpu.VMEM((_B2, D), jnp.float32),
            pltpu.SemaphoreType.DMA,
        ],
    )
    return f(mt, et, kj2, ji2)


_B4 = 512
_NBLK4 = A // _B4
_RSC = 25344
_TRASH = 128
_SROWS = _RSC + _TRASH
_NPASS = -(-E // (2 * _RSC))
_RING = 256
_FB = 128
_Q = 2


def _k4_body(ji2_hbm, em_hbm, agg_hbm, jib, emb, posf, dstf, snap, shared,
             sem_i, sem_g, sem_s):
    c = lax.axis_index("c")
    s = lax.axis_index("s")
    nfull = _NBLK4 // 16
    rem = _NBLK4 % 16
    trips = jnp.where(s < rem, nfull + 1, nfull)
    iota = lax.broadcasted_iota(jnp.int32, (16,), 0)
    z16 = jnp.zeros((16,), jnp.float32)
    zi16 = jnp.zeros((16,), jnp.int32)

    def _wait_gather():
        pltpu.make_async_copy(em_hbm.at[snap.at[0, 0]],
                              emb.at[pl.ds(0, _FB)], sem_g).wait()

    def _wait_scatter():
        pltpu.make_async_copy(emb.at[pl.ds(0, _FB)],
                              shared.at[snap.at[0, 1]], sem_s).wait()

    def _scatter(q):
        pltpu.async_copy(emb.at[pl.ds(q * _FB, _FB)],
                         shared.at[snap.at[q, 1]], sem_s, add=True)

    def _flush(done):
        q = (done // _FB) & (_Q - 1)
        _wait_scatter()
        start = done & (_RING - 1)
        for cc in range(_FB // 16):
            sl = pl.ds(cc * 16, 16)
            snap[q, 0, sl] = posf[pl.ds(start + cc * 16, 16)]
            snap[q, 1, sl] = dstf[pl.ds(start + cc * 16, 16)]
        pltpu.async_copy(em_hbm.at[snap.at[q, 0]],
                         emb.at[pl.ds(q * _FB, _FB)], sem_g)
        _wait_gather()
        _scatter((q - (_Q - 1)) & (_Q - 1))

    @pl.loop(0, _NPASS)
    def _pass(p):
        lo = p * 2 * _RSC + c * _RSC

        @pl.loop(0, _Q * _FB)
        def _zr(r):
            for cc in range(INT // 16):
                emb[r, pl.ds(cc * 16, 16)] = z16

        share = _SROWS // 16
        sbase = s * share
        for k in range(share // 128):
            pltpu.sync_copy(emb.at[pl.ds(0, 128)],
                            shared.at[pl.ds(sbase + k * 128, 128)])
        tail = share % 128
        if tail:
            pltpu.sync_copy(emb.at[pl.ds(0, tail)],
                            shared.at[pl.ds(sbase + (share // 128) * 128, tail)])
        plsc.subcore_barrier()

        for par in range(_Q):
            for cc in range(_FB // 16):
                sl = pl.ds(cc * 16, 16)
                snap[par, 0, sl] = zi16
                snap[par, 1, sl] = _RSC + (par * _FB + cc * 16) + iota
        for par in range(_Q):
            _scatter(par)
        for par in range(1, _Q):
            pltpu.async_copy(em_hbm.at[snap.at[par, 0]],
                             emb.at[pl.ds(par * _FB, _FB)], sem_g)
        pltpu.async_copy(ji2_hbm.at[pl.ds(s * 4, 4)], jib.at[0], sem_i)
        pltpu.async_copy(ji2_hbm.at[pl.ds((s + 16) * 4, 4)], jib.at[1], sem_i)

        @pl.loop(0, trips,
                 init_carry=(jnp.zeros((16,), jnp.int32), jnp.int32(0)))
        def _blk(g, carry):
            off, done = carry
            blk = s + g * 16
            gpar = g & 1
            pltpu.make_async_copy(ji2_hbm.at[pl.ds(blk * 4, 4)],
                                  jib.at[gpar], sem_i).wait()
            base = blk * _B4
            for j in range(4):
                for cc in range(8):
                    idxv = jib[gpar, j, pl.ds(cc * 16, 16)]
                    d = idxv - lo
                    m = (d >= 0) & (d < _RSC)
                    mi = m.astype(jnp.int32)
                    slot = (off + plsc.cumsum(mi) - 1) & (_RING - 1)
                    posg = (base + j * 128 + cc * 16) + iota
                    plsc.store_scatter(posf, [slot], posg, mask=m)
                    plsc.store_scatter(dstf, [slot], d, mask=m)
                    off = off + plsc.all_reduce_population_count(m)
                    if cc % 8 == 7:
                        full = jnp.any(off - done >= _FB)

                        @pl.when(full)
                        def _():
                            _flush(done)

                        done = jnp.where(full, done + _FB, done)

            @pl.when(g + 2 < trips)
            def _():
                blk2 = s + (g + 2) * 16
                pltpu.async_copy(ji2_hbm.at[pl.ds(blk2 * 4, 4)],
                                 jib.at[gpar], sem_i)

            return off, done

        off, done = _blk
        for k in range(_FB // 16):
            slot = (off + k * 16 + iota) & (_RING - 1)
            plsc.store_scatter(posf, [slot], zi16, mask=None)
            plsc.store_scatter(dstf, [slot], _RSC + k * 16 + iota, mask=None)
        _flush(done)
        done = done + _FB
        for k in range(_Q - 1):
            _wait_gather()
            _scatter((done // _FB - (_Q - 1) + k) & (_Q - 1))
        for k in range(2 * _Q - 1):
            _wait_scatter()

        plsc.subcore_barrier()
        dshare = _RSC // 16
        pltpu.sync_copy(shared.at[pl.ds(s * dshare, dshare)],
                        agg_hbm.at[pl.ds(lo + s * dshare, dshare)])
        plsc.subcore_barrier()


def _k4_call(ji2, em):
    mesh = plsc.VectorSubcoreMesh(core_axis_name="c", subcore_axis_name="s")
    f = pl.kernel(
        _k4_body,
        out_type=jax.ShapeDtypeStruct((_NPASS * 2 * _RSC, INT), jnp.float32),
        mesh=mesh,
        compiler_params=pltpu.CompilerParams(needs_layout_passes=False,
                                             use_tc_tiling_on_sc=False),
        scratch_types=[
            pltpu.VMEM((2, 4, 128), jnp.int32),
            pltpu.VMEM((_Q * _FB, INT), jnp.float32),
            pltpu.VMEM((_RING,), jnp.int32),
            pltpu.VMEM((_RING,), jnp.int32),
            pltpu.VMEM((_Q, 2, _FB), jnp.int32),
            pltpu.VMEM_SHARED((_SROWS, INT), jnp.float32),
            pltpu.SemaphoreType.DMA,
            pltpu.SemaphoreType.DMA,
            pltpu.SemaphoreType.DMA,
        ],
    )
    return f(ji2, em)


@jax.jit
def kernel(m_ji, e_rbf, a_sbf, kj_idx, ji_idx,
           W_mkj, b_mkj, We1, We2, Wa1, Wa2, W_down, W_up):
    b2 = b_mkj.reshape(1, D)
    mt, et = pl.pallas_call(
        _k1_body,
        grid=(E // _BE,),
        in_specs=[_row_blocked(E, _BE, D), _row_blocked(E, _BE, 6),
                  _full((D, D)), _full((1, D)), _full((6, 8)), _full((8, D))],
        out_specs=[_row_blocked(E, _BE, D), _row_blocked(E, _BE, D)],
        out_shape=[jax.ShapeDtypeStruct((E, D), jnp.float32),
                   jax.ShapeDtypeStruct((E, D), jnp.float32)],
    )(m_ji, e_rbf, W_mkj, b2, We1, We2)

    kj2 = kj_idx.astype(jnp.int32).reshape(A // 128, 128)
    ji2 = ji_idx.astype(jnp.int32).reshape(A // 128, 128)

    h = _k2_call(mt, et, kj2, ji2)

    em = pl.pallas_call(
        _k3_body,
        grid=(A // _BA,),
        in_specs=[_row_blocked(A, _BA, D), _row_blocked(A, _BA, 42),
                  _full((42, 8)), _full((8, INT)), _full((D, INT))],
        out_specs=_row_blocked(A, _BA, INT),
        out_shape=jax.ShapeDtypeStruct((A, INT), jnp.float32),
    )(h, a_sbf, Wa1, Wa2, W_down)

    agg = _k4_call(ji2, em)

    out = pl.pallas_call(
        _k5_body,
        grid=(E // _BE,),
        in_specs=[_row_blocked(E, _BE, INT), _full((INT, D))],
        out_specs=_row_blocked(E, _BE, D),
        out_shape=jax.ShapeDtypeStruct((E, D), jnp.float32),
    )(agg, W_up)
    return out


# speedup vs baseline: 1.1603x; 1.1603x over previous
import functools

import jax
import jax.numpy as jnp
from jax import lax
from jax.experimental import pallas as pl
from jax.experimental.pallas import tpu as pltpu
from jax.experimental.pallas import tpu_sc as plsc

E = 320000
A = 640000
D = 128
INT = 64


_BE = 2560
_BA = 2560


def _k1_body(m_ref, e_ref, wm_ref, b_ref, we1_ref, we2_ref, mt_ref, et_ref):
    z = jnp.dot(m_ref[...], wm_ref[...], preferred_element_type=jnp.float32)
    z = z + b_ref[...]
    mt_ref[...] = z * jax.nn.sigmoid(z)
    e1 = jnp.dot(e_ref[...], we1_ref[...], preferred_element_type=jnp.float32)
    et_ref[...] = jnp.dot(e1, we2_ref[...], preferred_element_type=jnp.float32)


def _k3_body(h_ref, a_ref, wa1_ref, wa2_ref, wd_ref, em_ref):
    s = jnp.dot(h_ref[...], wd_ref[...], preferred_element_type=jnp.float32)
    s = s * jax.nn.sigmoid(s)
    a1 = jnp.dot(a_ref[...], wa1_ref[...], preferred_element_type=jnp.float32)
    ap = jnp.dot(a1, wa2_ref[...], preferred_element_type=jnp.float32)
    em_ref[...] = s * ap


def _k5_body(g_ref, wu_ref, o_ref):
    z = jnp.dot(g_ref[...], wu_ref[...], preferred_element_type=jnp.float32)
    o_ref[...] = z * jax.nn.sigmoid(z)


def _row_blocked(nrows, block, ncols):
    return pl.BlockSpec((block, ncols), lambda i: (i, 0))


def _full(shape):
    return pl.BlockSpec(shape, lambda i: tuple(0 for _ in shape))


_NW = 32
_B2 = 256
_NBLK2 = A // _B2


def _k2_body(mt_hbm, et_hbm, kj2_hbm, ji2_hbm, h_hbm,
             kjb, jib, mrows, erows, sem):
    c = lax.axis_index("c")
    s = lax.axis_index("s")
    wid = s * 2 + c
    nfull = _NBLK2 // _NW
    rem = _NBLK2 % _NW
    trips = jnp.where(wid < rem, nfull + 1, nfull)

    @pl.loop(0, trips)
    def _block(g):
        blk = wid + g * _NW
        base = blk * _B2
        pltpu.sync_copy(kj2_hbm.at[pl.ds(blk * 2, 2)], kjb)
        pltpu.sync_copy(ji2_hbm.at[pl.ds(blk * 2, 2)], jib)
        descs = []
        for j in range(2):
            descs.append(pltpu.async_copy(
                mt_hbm.at[kjb.at[j]], mrows.at[pl.ds(j * 128, 128)], sem))
            descs.append(pltpu.async_copy(
                et_hbm.at[jib.at[j]], erows.at[pl.ds(j * 128, 128)], sem))
        for d in descs:
            d.wait()

        @pl.loop(0, _B2)
        def _row(r):
            for cc in range(D // 16):
                sl = pl.ds(cc * 16, 16)
                mrows[r, sl] = mrows[r, sl] * erows[r, sl]

        pltpu.sync_copy(mrows, h_hbm.at[pl.ds(base, _B2)])


def _k2_call(mt, et, kj2, ji2):
    mesh = plsc.VectorSubcoreMesh(core_axis_name="c", subcore_axis_name="s")
    f = pl.kernel(
        _k2_body,
        out_type=jax.ShapeDtypeStruct((A, D), jnp.float32),
        mesh=mesh,
        compiler_params=pltpu.CompilerParams(needs_layout_passes=False),
        scratch_types=[
            pltpu.VMEM((2, 128), jnp.int32),
            pltpu.VMEM((2, 128), jnp.int32),
            pltpu.VMEM((_B2, D), jnp.float32),
            pltpu.VMEM((_B2, D), jnp.float32),
            pltpu.SemaphoreType.DMA,
        ],
    )
    return f(mt, et, kj2, ji2)


_B4 = 512
_NBLK4 = A // _B4
_RSC = 27392
_TRASH = 128
_SROWS = _RSC + _TRASH
_NPASS = -(-E // (2 * _RSC))
_RING = 256
_FB = 64
_Q = 2


def _k4_body(ji2_hbm, em_hbm, agg_hbm, jib, emb, posf, dstf, snap, shared,
             sem_i, sem_g, sem_s):
    c = lax.axis_index("c")
    s = lax.axis_index("s")
    nfull = _NBLK4 // 16
    rem = _NBLK4 % 16
    trips = jnp.where(s < rem, nfull + 1, nfull)
    iota = lax.broadcasted_iota(jnp.int32, (16,), 0)
    z16 = jnp.zeros((16,), jnp.float32)
    zi16 = jnp.zeros((16,), jnp.int32)

    def _wait_gather():
        pltpu.make_async_copy(em_hbm.at[snap.at[0, 0]],
                              emb.at[pl.ds(0, _FB)], sem_g).wait()

    def _wait_scatter():
        pltpu.make_async_copy(emb.at[pl.ds(0, _FB)],
                              shared.at[snap.at[0, 1]], sem_s).wait()

    def _scatter(q):
        pltpu.async_copy(emb.at[pl.ds(q * _FB, _FB)],
                         shared.at[snap.at[q, 1]], sem_s, add=True)

    def _flush(done):
        q = (done // _FB) & (_Q - 1)
        _wait_scatter()
        start = done & (_RING - 1)
        for cc in range(_FB // 16):
            sl = pl.ds(cc * 16, 16)
            snap[q, 0, sl] = posf[pl.ds(start + cc * 16, 16)]
            snap[q, 1, sl] = dstf[pl.ds(start + cc * 16, 16)]
        pltpu.async_copy(em_hbm.at[snap.at[q, 0]],
                         emb.at[pl.ds(q * _FB, _FB)], sem_g)
        _wait_gather()
        _scatter((q - (_Q - 1)) & (_Q - 1))

    @pl.loop(0, _NPASS)
    def _pass(p):
        lo = p * 2 * _RSC + c * _RSC

        @pl.loop(0, _Q * _FB)
        def _zr(r):
            for cc in range(INT // 16):
                emb[r, pl.ds(cc * 16, 16)] = z16

        share = _SROWS // 16
        sbase = s * share
        for k in range(share // 128):
            pltpu.sync_copy(emb, shared.at[pl.ds(sbase + k * 128, 128)])
        tail = share % 128
        if tail:
            pltpu.sync_copy(emb.at[pl.ds(0, tail)],
                            shared.at[pl.ds(sbase + (share // 128) * 128, tail)])
        plsc.subcore_barrier()

        for par in range(_Q):
            for cc in range(_FB // 16):
                sl = pl.ds(cc * 16, 16)
                snap[par, 0, sl] = zi16
                snap[par, 1, sl] = _RSC + (par * _FB + cc * 16) + iota
        for par in range(_Q):
            _scatter(par)
        for par in range(1, _Q):
            pltpu.async_copy(em_hbm.at[snap.at[par, 0]],
                             emb.at[pl.ds(par * _FB, _FB)], sem_g)
        pltpu.async_copy(ji2_hbm.at[pl.ds(s * 4, 4)], jib.at[0], sem_i)
        pltpu.async_copy(ji2_hbm.at[pl.ds((s + 16) * 4, 4)], jib.at[1], sem_i)

        @pl.loop(0, trips,
                 init_carry=(jnp.zeros((16,), jnp.int32), jnp.int32(0)))
        def _blk(g, carry):
            off, done = carry
            blk = s + g * 16
            gpar = g & 1
            pltpu.make_async_copy(ji2_hbm.at[pl.ds(blk * 4, 4)],
                                  jib.at[gpar], sem_i).wait()
            base = blk * _B4
            for j in range(4):
                for cc in range(8):
                    idxv = jib[gpar, j, pl.ds(cc * 16, 16)]
                    d = idxv - lo
                    m = (d >= 0) & (d < _RSC)
                    mi = m.astype(jnp.int32)
                    slot = (off + plsc.cumsum(mi) - 1) & (_RING - 1)
                    posg = (base + j * 128 + cc * 16) + iota
                    plsc.store_scatter(posf, [slot], posg, mask=m)
                    plsc.store_scatter(dstf, [slot], d, mask=m)
                    off = off + plsc.all_reduce_population_count(m)
                    if cc % 4 == 3:
                        full = jnp.any(off - done >= _FB)

                        @pl.when(full)
                        def _():
                            _flush(done)

                        done = jnp.where(full, done + _FB, done)

            @pl.when(g + 2 < trips)
            def _():
                blk2 = s + (g + 2) * 16
                pltpu.async_copy(ji2_hbm.at[pl.ds(blk2 * 4, 4)],
                                 jib.at[gpar], sem_i)

            return off, done

        off, done = _blk
        for k in range(_FB // 16):
            slot = (off + k * 16 + iota) & (_RING - 1)
            plsc.store_scatter(posf, [slot], zi16, mask=None)
            plsc.store_scatter(dstf, [slot], _RSC + k * 16 + iota, mask=None)
        _flush(done)
        done = done + _FB
        for k in range(_Q - 1):
            _wait_gather()
            _scatter((done // _FB - (_Q - 1) + k) & (_Q - 1))
        for k in range(2 * _Q - 1):
            _wait_scatter()

        plsc.subcore_barrier()
        dshare = _RSC // 16
        pltpu.sync_copy(shared.at[pl.ds(s * dshare, dshare)],
                        agg_hbm.at[pl.ds(lo + s * dshare, dshare)])
        plsc.subcore_barrier()


def _k4_call(ji2, em):
    mesh = plsc.VectorSubcoreMesh(core_axis_name="c", subcore_axis_name="s")
    f = pl.kernel(
        _k4_body,
        out_type=jax.ShapeDtypeStruct((_NPASS * 2 * _RSC, INT), jnp.float32),
        mesh=mesh,
        compiler_params=pltpu.CompilerParams(needs_layout_passes=False,
                                             use_tc_tiling_on_sc=False),
        scratch_types=[
            pltpu.VMEM((2, 4, 128), jnp.int32),
            pltpu.VMEM((_Q * _FB, INT), jnp.float32),
            pltpu.VMEM((_RING,), jnp.int32),
            pltpu.VMEM((_RING,), jnp.int32),
            pltpu.VMEM((_Q, 2, _FB), jnp.int32),
            pltpu.VMEM_SHARED((_SROWS, INT), jnp.float32),
            pltpu.SemaphoreType.DMA,
            pltpu.SemaphoreType.DMA,
            pltpu.SemaphoreType.DMA,
        ],
    )
    return f(ji2, em)


@jax.jit
def kernel(m_ji, e_rbf, a_sbf, kj_idx, ji_idx,
           W_mkj, b_mkj, We1, We2, Wa1, Wa2, W_down, W_up):
    b2 = b_mkj.reshape(1, D)
    mt, et = pl.pallas_call(
        _k1_body,
        grid=(E // _BE,),
        in_specs=[_row_blocked(E, _BE, D), _row_blocked(E, _BE, 6),
                  _full((D, D)), _full((1, D)), _full((6, 8)), _full((8, D))],
        out_specs=[_row_blocked(E, _BE, D), _row_blocked(E, _BE, D)],
        out_shape=[jax.ShapeDtypeStruct((E, D), jnp.float32),
                   jax.ShapeDtypeStruct((E, D), jnp.float32)],
    )(m_ji, e_rbf, W_mkj, b2, We1, We2)

    kj2 = kj_idx.astype(jnp.int32).reshape(A // 128, 128)
    ji2 = ji_idx.astype(jnp.int32).reshape(A // 128, 128)

    h = _k2_call(mt, et, kj2, ji2)

    em = pl.pallas_call(
        _k3_body,
        grid=(A // _BA,),
        in_specs=[_row_blocked(A, _BA, D), _row_blocked(A, _BA, 42),
                  _full((42, 8)), _full((8, INT)), _full((D, INT))],
        out_specs=_row_blocked(A, _BA, INT),
        out_shape=jax.ShapeDtypeStruct((A, INT), jnp.float32),
    )(h, a_sbf, Wa1, Wa2, W_down)

    agg = _k4_call(ji2, em)

    out = pl.pallas_call(
        _k5_body,
        grid=(E // _BE,),
        in_specs=[_row_blocked(E, _BE, INT), _full((INT, D))],
        out_specs=_row_blocked(E, _BE, D),
        out_shape=jax.ShapeDtypeStruct((E, D), jnp.float32),
    )(agg, W_up)
    return out
